# baseline (device time: 315405 ns/iter reference)
import functools

import jax
import jax.numpy as jnp
from jax import lax
from jax.experimental import pallas as pl
from jax.experimental.pallas import tpu as pltpu

N_DEV = 16
SQ = 2048
D_MODEL = 1024
H_TOTAL = 128
H_PER = 8
DH = 128
CHUNK = SQ // N_DEV
N_HOPS = N_DEV - 1
SCALE = 0.08838834764831843


def _partial_output(x, Wq, K_ext, V_ext, Wo):
    me = lax.axis_index("i")
    bf = jnp.bfloat16
    xb = x.reshape(SQ, D_MODEL).astype(bf)
    Q = jnp.dot(xb, Wq.astype(bf), preferred_element_type=jnp.float32)
    Q = Q.reshape(SQ, H_PER, DH)
    K = lax.dynamic_slice_in_dim(
        K_ext.reshape(SQ, H_TOTAL, DH), me * H_PER, H_PER, axis=1
    )
    V = lax.dynamic_slice_in_dim(
        V_ext.reshape(SQ, H_TOTAL, DH), me * H_PER, H_PER, axis=1
    )

    def group(t):
        return (
            t.reshape(8, 4, 64, H_PER, DH)
            .transpose(1, 0, 2, 3, 4)
            .reshape(4, 512, H_PER, DH)
        )

    Qp = group(Q).astype(bf)
    Kp = group(K.astype(bf))
    Vp = group(V.astype(bf))
    s = jnp.einsum("rqhd,rkhd->rhqk", Qp, Kp, preferred_element_type=jnp.float32)
    w = jax.nn.softmax(s * SCALE, axis=-1)
    ctx = jnp.einsum(
        "rhqk,rkhd->rqhd", w.astype(bf), Vp, preferred_element_type=jnp.float32
    )
    ctx = (
        ctx.reshape(4, 8, 64, H_PER, DH)
        .transpose(1, 0, 2, 3, 4)
        .reshape(SQ, H_PER * DH)
    )
    return jnp.dot(ctx.astype(bf), Wo.astype(bf), preferred_element_type=jnp.float32)


def _ring_all_reduce(partial):

    def body(
        p_ref,
        out_ref,
        send_buf,
        own_buf,
        rs_buf,
        ag_buf,
        rs_send_sems,
        rs_recv_sems,
        ag_send_sems,
        ag_recv_sems,
    ):
        me = lax.axis_index("i")
        left = lax.rem(me + N_DEV - 1, N_DEV)
        right = lax.rem(me + 1, N_DEV)

        barrier = pltpu.get_barrier_semaphore()
        for nbr in (left, right):
            pl.semaphore_signal(
                barrier, inc=1, device_id=(nbr,),
                device_id_type=pl.DeviceIdType.MESH,
            )
        pl.semaphore_wait(barrier, 2)

        def chunk(ref, idx):
            return ref[pl.ds(idx * CHUNK, CHUNK), :]

        send_buf[0] = chunk(p_ref, me)
        for h in range(N_HOPS):
            slot = h % 2
            if h >= 1:
                c = lax.rem(me - h + N_DEV, N_DEV)
                send_buf[slot] = chunk(p_ref, c) + rs_buf[h - 1]
            rdma = pltpu.make_async_remote_copy(
                src_ref=send_buf.at[slot],
                dst_ref=rs_buf.at[h],
                send_sem=rs_send_sems.at[slot],
                recv_sem=rs_recv_sems.at[h],
                device_id=(right,),
                device_id_type=pl.DeviceIdType.MESH,
            )
            rdma.start()
            rdma.wait()

        own = lax.rem(me + 1, N_DEV)
        own_buf[:, :] = chunk(p_ref, own) + rs_buf[N_HOPS - 1]
        out_ref[pl.ds(own * CHUNK, CHUNK), :] = own_buf[:, :]

        for h in range(N_HOPS):
            src = own_buf if h == 0 else ag_buf.at[h - 1]
            rdma = pltpu.make_async_remote_copy(
                src_ref=src,
                dst_ref=ag_buf.at[h],
                send_sem=ag_send_sems.at[h % 2],
                recv_sem=ag_recv_sems.at[h],
                device_id=(right,),
                device_id_type=pl.DeviceIdType.MESH,
            )
            rdma.start()
            rdma.wait()
            r = lax.rem(me - h + N_DEV, N_DEV)
            out_ref[pl.ds(r * CHUNK, CHUNK), :] = ag_buf[h]

        @functools.partial(pl.run_scoped, sem=pltpu.SemaphoreType.REGULAR)
        def _(sem):
            for nbr in (left, right):
                pl.semaphore_signal(
                    sem, inc=1, device_id=(nbr,),
                    device_id_type=pl.DeviceIdType.MESH,
                )
            pl.semaphore_wait(sem, 2)

    return pl.pallas_call(
        body,
        out_shape=jax.ShapeDtypeStruct((SQ, D_MODEL), jnp.float32),
        in_specs=[pl.BlockSpec(memory_space=pltpu.VMEM)],
        out_specs=pl.BlockSpec(memory_space=pltpu.VMEM),
        scratch_shapes=[
            pltpu.VMEM((2, CHUNK, D_MODEL), jnp.float32),
            pltpu.VMEM((CHUNK, D_MODEL), jnp.float32),
            pltpu.VMEM((N_HOPS, CHUNK, D_MODEL), jnp.float32),
            pltpu.VMEM((N_HOPS, CHUNK, D_MODEL), jnp.float32),
            pltpu.SemaphoreType.DMA((2,)),
            pltpu.SemaphoreType.DMA((N_HOPS,)),
            pltpu.SemaphoreType.DMA((2,)),
            pltpu.SemaphoreType.DMA((N_HOPS,)),
        ],
        compiler_params=pltpu.CompilerParams(collective_id=0),
    )(partial)


def kernel(x, Wq, K_ext, V_ext, Wo):
    part = _partial_output(x, Wq, K_ext, V_ext, Wo)
    out = _ring_all_reduce(part)
    return out.reshape(1, SQ, D_MODEL)


# device time: 245774 ns/iter; 1.2833x vs baseline; 1.2833x over previous
import functools

import jax
import jax.numpy as jnp
from jax import lax
from jax.experimental import pallas as pl
from jax.experimental.pallas import tpu as pltpu

N_DEV = 16
SQ = 2048
D_MODEL = 1024
H_TOTAL = 128
H_PER = 8
DH = 128
CHUNK = SQ // N_DEV
N_HOPS = N_DEV - 1
SCALE = 0.08838834764831843


def _partial_output(x, Wq, K_ext, V_ext, Wo):
    me = lax.axis_index("i")
    bf = jnp.bfloat16
    xb = x.reshape(SQ, D_MODEL).astype(bf)
    Q = jnp.dot(xb, Wq.astype(bf), preferred_element_type=jnp.float32)
    Q = Q.reshape(SQ, H_PER, DH)
    K = lax.dynamic_slice_in_dim(
        K_ext.reshape(SQ, H_TOTAL, DH), me * H_PER, H_PER, axis=1
    )
    V = lax.dynamic_slice_in_dim(
        V_ext.reshape(SQ, H_TOTAL, DH), me * H_PER, H_PER, axis=1
    )

    def group(t):
        return (
            t.reshape(8, 4, 64, H_PER, DH)
            .transpose(1, 0, 2, 3, 4)
            .reshape(4, 512, H_PER, DH)
        )

    Qp = group(Q).astype(bf)
    Kp = group(K.astype(bf))
    Vp = group(V.astype(bf))
    s = jnp.einsum("rqhd,rkhd->rhqk", Qp, Kp, preferred_element_type=jnp.float32)
    w = jax.nn.softmax(s * SCALE, axis=-1)
    ctx = jnp.einsum(
        "rhqk,rkhd->rqhd", w.astype(bf), Vp, preferred_element_type=jnp.float32
    )
    ctx = (
        ctx.reshape(4, 8, 64, H_PER, DH)
        .transpose(1, 0, 2, 3, 4)
        .reshape(SQ, H_PER * DH)
    )
    return jnp.dot(ctx.astype(bf), Wo.astype(bf), preferred_element_type=jnp.float32)


HALF = D_MODEL // 2


def _ring_all_reduce(partial):

    def body(
        p_ref,
        out_ref,
        send_buf,
        own_buf,
        rs_buf,
        ag_buf,
        rs_send_sems,
        rs_recv_sems,
        ag_send_sems,
        ag_recv_sems,
    ):
        me = lax.axis_index("i")
        left = lax.rem(me + N_DEV - 1, N_DEV)
        right = lax.rem(me + 1, N_DEV)

        barrier = pltpu.get_barrier_semaphore()
        for nbr in (left, right):
            pl.semaphore_signal(
                barrier, inc=1, device_id=(nbr,),
                device_id_type=pl.DeviceIdType.MESH,
            )
        pl.semaphore_wait(barrier, 2)

        def chunk(idx, d):
            return p_ref[pl.ds(idx * CHUNK, CHUNK), pl.ds(d * HALF, HALF)]

        def rs_chunk_idx(h, d):
            return lax.rem(me + (h if d else -h) + N_DEV, N_DEV)

        dst_dev = (right, left)

        send_buf[0, 0] = chunk(me, 0)
        send_buf[0, 1] = chunk(me, 1)
        for h in range(N_HOPS):
            slot = h % 2
            rdmas = []
            for d in (0, 1):
                if h >= 1:
                    send_buf[slot, d] = chunk(rs_chunk_idx(h, d), d) + rs_buf[d, h - 1]
                rdma = pltpu.make_async_remote_copy(
                    src_ref=send_buf.at[slot, d],
                    dst_ref=rs_buf.at[d, h],
                    send_sem=rs_send_sems.at[slot, d],
                    recv_sem=rs_recv_sems.at[d, h],
                    device_id=(dst_dev[d],),
                    device_id_type=pl.DeviceIdType.MESH,
                )
                rdma.start()
                rdmas.append(rdma)
            for rdma in rdmas:
                rdma.wait()

        own = (right, left)
        for d in (0, 1):
            own_buf[d] = (chunk(own[d], d) + rs_buf[d, N_HOPS - 1]).astype(
                jnp.bfloat16
            )
            out_ref[pl.ds(own[d] * CHUNK, CHUNK), pl.ds(d * HALF, HALF)] = (
                own_buf[d].astype(jnp.float32)
            )

        for h in range(N_HOPS):
            rdmas = []
            for d in (0, 1):
                src = own_buf.at[d] if h == 0 else ag_buf.at[d, h - 1]
                rdma = pltpu.make_async_remote_copy(
                    src_ref=src,
                    dst_ref=ag_buf.at[d, h],
                    send_sem=ag_send_sems.at[h % 2, d],
                    recv_sem=ag_recv_sems.at[d, h],
                    device_id=(dst_dev[d],),
                    device_id_type=pl.DeviceIdType.MESH,
                )
                rdma.start()
                rdmas.append(rdma)
            for rdma in rdmas:
                rdma.wait()
            for d in (0, 1):
                r = lax.rem(me + (h if d else -h) + N_DEV, N_DEV)
                out_ref[pl.ds(r * CHUNK, CHUNK), pl.ds(d * HALF, HALF)] = (
                    ag_buf[d, h].astype(jnp.float32)
                )

        @functools.partial(pl.run_scoped, sem=pltpu.SemaphoreType.REGULAR)
        def _(sem):
            for nbr in (left, right):
                pl.semaphore_signal(
                    sem, inc=1, device_id=(nbr,),
                    device_id_type=pl.DeviceIdType.MESH,
                )
            pl.semaphore_wait(sem, 2)

    return pl.pallas_call(
        body,
        out_shape=jax.ShapeDtypeStruct((SQ, D_MODEL), jnp.float32),
        in_specs=[pl.BlockSpec(memory_space=pltpu.VMEM)],
        out_specs=pl.BlockSpec(memory_space=pltpu.VMEM),
        scratch_shapes=[
            pltpu.VMEM((2, 2, CHUNK, HALF), jnp.float32),
            pltpu.VMEM((2, CHUNK, HALF), jnp.bfloat16),
            pltpu.VMEM((2, N_HOPS, CHUNK, HALF), jnp.float32),
            pltpu.VMEM((2, N_HOPS, CHUNK, HALF), jnp.bfloat16),
            pltpu.SemaphoreType.DMA((2, 2)),
            pltpu.SemaphoreType.DMA((2, N_HOPS)),
            pltpu.SemaphoreType.DMA((2, 2)),
            pltpu.SemaphoreType.DMA((2, N_HOPS)),
        ],
        compiler_params=pltpu.CompilerParams(collective_id=0),
    )(partial)


def kernel(x, Wq, K_ext, V_ext, Wo):
    part = _partial_output(x, Wq, K_ext, V_ext, Wo)
    out = _ring_all_reduce(part)
    return out.reshape(1, SQ, D_MODEL)


# device time: 224945 ns/iter; 1.4021x vs baseline; 1.0926x over previous
import functools

import jax
import jax.numpy as jnp
from jax import lax
from jax.experimental import pallas as pl
from jax.experimental.pallas import tpu as pltpu

N_DEV = 16
SQ = 2048
D_MODEL = 1024
H_TOTAL = 128
H_PER = 8
DH = 128
CHUNK = SQ // N_DEV
N_HOPS = N_DEV - 1
HALF = D_MODEL // 2
SCALE = 0.08838834764831843


def _group_rows(t):
    return (
        t.reshape(8, 4, 64, *t.shape[1:])
        .transpose(1, 0, 2, *range(3, t.ndim + 2))
        .reshape(4 * 512, *t.shape[1:])
    )


def _prep(x, Wq, K_ext, V_ext, Wo):
    me = lax.axis_index("i")
    bf = jnp.bfloat16
    xg = _group_rows(x.reshape(SQ, D_MODEL)).astype(bf)
    K = lax.dynamic_slice_in_dim(
        K_ext.reshape(SQ, H_TOTAL, DH), me * H_PER, H_PER, axis=1
    )
    V = lax.dynamic_slice_in_dim(
        V_ext.reshape(SQ, H_TOTAL, DH), me * H_PER, H_PER, axis=1
    )
    Kg = _group_rows(K).reshape(4, 512, H_PER, DH)
    Vg = _group_rows(V).reshape(4, 512, H_PER, DH)
    KgT = Kg.transpose(0, 2, 3, 1).astype(bf)
    Vgt = Vg.transpose(0, 2, 1, 3).astype(bf)
    Wq_b = Wq.astype(bf)
    Wo_r = Wo.reshape(H_PER, DH, D_MODEL).astype(bf)
    return xg, Wq_b, KgT, Vgt, Wo_r


def _fused(xg, Wq_b, KgT, Vgt, Wo_r):
    def body(
        xg_ref,
        wq_ref,
        kgt_ref,
        vgt_ref,
        wo_ref,
        out_ref,
        part_ref,
        send_buf,
        own_buf,
        rs_buf,
        ag_buf,
        rs_send_sems,
        rs_recv_sems,
        ag_send_sems,
        ag_recv_sems,
    ):
        me = lax.axis_index("i")
        left = lax.rem(me + N_DEV - 1, N_DEV)
        right = lax.rem(me + 1, N_DEV)
        bf = jnp.bfloat16

        barrier = pltpu.get_barrier_semaphore()
        for nbr in (left, right):
            pl.semaphore_signal(
                barrier, inc=1, device_id=(nbr,),
                device_id_type=pl.DeviceIdType.MESH,
            )
        pl.semaphore_wait(barrier, 2)

        def compute_chunk(c):
            r = c // 4
            xc = xg_ref[pl.ds(c * CHUNK, CHUNK), :]
            qc = jnp.dot(xc, wq_ref[:, :], preferred_element_type=jnp.float32)
            qc = qc.astype(bf)
            acc = jnp.zeros((CHUNK, D_MODEL), jnp.float32)
            for h in range(H_PER):
                qh = qc[:, h * DH : (h + 1) * DH]
                kh = kgt_ref[r, h]
                s = jnp.dot(qh, kh, preferred_element_type=jnp.float32) * SCALE
                m = jnp.max(s, axis=-1, keepdims=True)
                e = jnp.exp(s - m)
                w = (e / jnp.sum(e, axis=-1, keepdims=True)).astype(bf)
                ctx = jnp.dot(
                    w, vgt_ref[r, h], preferred_element_type=jnp.float32
                ).astype(bf)
                acc = acc + jnp.dot(
                    ctx, wo_ref[h], preferred_element_type=jnp.float32
                )
            part_ref[pl.ds(c * CHUNK, CHUNK), :] = acc

        def store_chunk(c, d, data):
            j = lax.rem(c, 4)
            r = c // 4
            b0 = 8 * j + r
            col = pl.ds(d * HALF, HALF)
            out_ref[pl.ds(b0 * 64, 64), col] = data[:64]
            out_ref[pl.ds(b0 * 64 + 256, 64), col] = data[64:]

        dst_dev = (right, left)

        def rs_rdma(h, slot, d):
            return pltpu.make_async_remote_copy(
                src_ref=send_buf.at[slot, d],
                dst_ref=rs_buf.at[d, h],
                send_sem=rs_send_sems.at[slot, d],
                recv_sem=rs_recv_sems.at[d, h],
                device_id=(dst_dev[d],),
                device_id_type=pl.DeviceIdType.MESH,
            )

        def part_half(c, d):
            return part_ref[pl.ds(c * CHUNK, CHUNK), pl.ds(d * HALF, HALF)]

        compute_chunk(me)
        rdmas = []
        for d in (0, 1):
            send_buf[0, d] = part_half(me, d)
            rdma = rs_rdma(0, 0, d)
            rdma.start()
            rdmas.append(rdma)
        for h in range(1, N_DEV):
            if h <= 8:
                compute_chunk(lax.rem(me - h + N_DEV, N_DEV))
            if h <= 7:
                compute_chunk(lax.rem(me + h, N_DEV))
            for rdma in rdmas:
                rdma.wait()
            if h <= N_HOPS - 1:
                rdmas = []
                slot = h % 2
                for d in (0, 1):
                    c = lax.rem(me + (h if d else -h) + N_DEV, N_DEV)
                    send_buf[slot, d] = part_half(c, d) + rs_buf[d, h - 1]
                    rdma = rs_rdma(h, slot, d)
                    rdma.start()
                    rdmas.append(rdma)

        own = (right, left)
        for d in (0, 1):
            own_f32 = part_half(own[d], d) + rs_buf[d, N_HOPS - 1]
            own_buf[d] = own_f32.astype(bf)
            store_chunk(own[d], d, own_f32)

        def ag_rdma(h, d):
            src = own_buf.at[d] if h == 0 else ag_buf.at[d, h - 1]
            return pltpu.make_async_remote_copy(
                src_ref=src,
                dst_ref=ag_buf.at[d, h],
                send_sem=ag_send_sems.at[h % 2, d],
                recv_sem=ag_recv_sems.at[d, h],
                device_id=(dst_dev[d],),
                device_id_type=pl.DeviceIdType.MESH,
            )

        rdmas = [ag_rdma(0, 0), ag_rdma(0, 1)]
        for r_ in rdmas:
            r_.start()
        for h in range(N_HOPS):
            for r_ in rdmas:
                r_.wait()
            if h + 1 < N_HOPS:
                rdmas = [ag_rdma(h + 1, 0), ag_rdma(h + 1, 1)]
                for r_ in rdmas:
                    r_.start()
            for d in (0, 1):
                c = lax.rem(me + (h if d else -h) + N_DEV, N_DEV)
                store_chunk(c, d, ag_buf[d, h].astype(jnp.float32))

        @functools.partial(pl.run_scoped, sem=pltpu.SemaphoreType.REGULAR)
        def _(sem):
            for nbr in (left, right):
                pl.semaphore_signal(
                    sem, inc=1, device_id=(nbr,),
                    device_id_type=pl.DeviceIdType.MESH,
                )
            pl.semaphore_wait(sem, 2)

    return pl.pallas_call(
        body,
        out_shape=jax.ShapeDtypeStruct((SQ, D_MODEL), jnp.float32),
        in_specs=[pl.BlockSpec(memory_space=pltpu.VMEM)] * 5,
        out_specs=pl.BlockSpec(memory_space=pltpu.VMEM),
        scratch_shapes=[
            pltpu.VMEM((SQ, D_MODEL), jnp.float32),
            pltpu.VMEM((2, 2, CHUNK, HALF), jnp.float32),
            pltpu.VMEM((2, CHUNK, HALF), jnp.bfloat16),
            pltpu.VMEM((2, N_HOPS, CHUNK, HALF), jnp.float32),
            pltpu.VMEM((2, N_HOPS, CHUNK, HALF), jnp.bfloat16),
            pltpu.SemaphoreType.DMA((2, 2)),
            pltpu.SemaphoreType.DMA((2, N_HOPS)),
            pltpu.SemaphoreType.DMA((2, 2)),
            pltpu.SemaphoreType.DMA((2, N_HOPS)),
        ],
        compiler_params=pltpu.CompilerParams(collective_id=0),
    )(xg, Wq_b, KgT, Vgt, Wo_r)


def kernel(x, Wq, K_ext, V_ext, Wo):
    out = _fused(*_prep(x, Wq, K_ext, V_ext, Wo))
    return out.reshape(1, SQ, D_MODEL)


# device time: 200101 ns/iter; 1.5762x vs baseline; 1.1242x over previous
import functools

import jax
import jax.numpy as jnp
from jax import lax
from jax.experimental import pallas as pl
from jax.experimental.pallas import tpu as pltpu

N_DEV = 16
SQ = 2048
D_MODEL = 1024
H_TOTAL = 128
H_PER = 8
DH = 128
CHUNK = SQ // N_DEV
QROWS = SQ // 4
HALF = D_MODEL // 2
SCALE = 0.08838834764831843


def _group_rows(t):
    return (
        t.reshape(8, 4, 64, *t.shape[1:])
        .transpose(1, 0, 2, *range(3, t.ndim + 2))
        .reshape(4 * 512, *t.shape[1:])
    )


def _prep(x, Wq, K_ext, V_ext, Wo):
    me = lax.axis_index("i")
    bf = jnp.bfloat16
    xg = _group_rows(x.reshape(SQ, D_MODEL)).astype(bf)
    K = lax.dynamic_slice_in_dim(
        K_ext.reshape(SQ, H_TOTAL, DH), me * H_PER, H_PER, axis=1
    )
    V = lax.dynamic_slice_in_dim(
        V_ext.reshape(SQ, H_TOTAL, DH), me * H_PER, H_PER, axis=1
    )
    Kg = _group_rows(K).reshape(4, 512, H_PER, DH)
    Vg = _group_rows(V).reshape(4, 512, H_PER, DH)
    KgT = Kg.transpose(0, 2, 3, 1).astype(bf)
    Vgt = Vg.transpose(0, 2, 1, 3).astype(bf)
    Wq_b = Wq.astype(bf)
    Wo_r = Wo.reshape(H_PER, DH, D_MODEL).astype(bf)
    return xg, Wq_b, KgT, Vgt, Wo_r


def _fused(xg, Wq_b, KgT, Vgt, Wo_r):
    def body(
        xg_ref,
        wq_ref,
        kgt_ref,
        vgt_ref,
        wo_ref,
        out_ref,
        part_ref,
        sbB,
        rbB,
        accB,
        sbA,
        rbA,
        ownA,
        qbuf,
        rbC,
        rbD,
        semB_send,
        semB_recv,
        semA_send,
        semA_recv,
        semC_send,
        semC_recv,
        semD_send,
        semD_recv,
    ):
        me = lax.axis_index("i")
        bf = jnp.bfloat16
        q = lax.rem(me, 4)
        zi = me // 4
        pbase = me - q
        ipn = pbase + lax.rem(q + 1, 4)
        ipp = pbase + lax.rem(q + 3, 4)
        cn = lax.rem(zi + 1, 4) * 4 + q
        cp = lax.rem(zi + 3, 4) * 4 + q
        plane_tgt = (ipn, ipp)
        col_tgt = (cn, cp)

        barrier = pltpu.get_barrier_semaphore()
        for nbr in (ipn, ipp, cn, cp):
            pl.semaphore_signal(
                barrier, inc=1, device_id=(nbr,),
                device_id_type=pl.DeviceIdType.MESH,
            )
        pl.semaphore_wait(barrier, 4)

        def compute_chunk(c):
            r = c // 4
            xc = xg_ref[pl.ds(c * CHUNK, CHUNK), :]
            qc = jnp.dot(xc, wq_ref[:, :], preferred_element_type=jnp.float32)
            qc = qc.astype(bf)
            acc = jnp.zeros((CHUNK, D_MODEL), jnp.float32)
            for h in range(H_PER):
                qh = qc[:, h * DH : (h + 1) * DH]
                kh = kgt_ref[r, h]
                s = jnp.dot(qh, kh, preferred_element_type=jnp.float32) * SCALE
                m = jnp.max(s, axis=-1, keepdims=True)
                e = jnp.exp(s - m)
                w = (e / jnp.sum(e, axis=-1, keepdims=True)).astype(bf)
                ctx = jnp.dot(
                    w, vgt_ref[r, h], preferred_element_type=jnp.float32
                ).astype(bf)
                acc = acc + jnp.dot(
                    ctx, wo_ref[h], preferred_element_type=jnp.float32
                )
            part_ref[pl.ds(c * CHUNK, CHUNK), :] = acc

        def store_chunk(c, d, data):
            j = lax.rem(c, 4)
            r = c // 4
            b0 = 8 * j + r
            col = pl.ds(d * HALF, HALF)
            out_ref[pl.ds(b0 * 64, 64), col] = data[:64]
            out_ref[pl.ds(b0 * 64 + 256, 64), col] = data[64:]

        def part_q(Q, d):
            return part_ref[pl.ds(Q * QROWS, QROWS), pl.ds(d * HALF, HALF)]

        def accB_z(Z, d):
            return accB[d, pl.ds(Z * CHUNK, CHUNK), :]

        def b_rdma(h, slot, d):
            return pltpu.make_async_remote_copy(
                src_ref=sbB.at[slot, d],
                dst_ref=rbB.at[d, h],
                send_sem=semB_send.at[slot, d],
                recv_sem=semB_recv.at[d, h],
                device_id=(plane_tgt[d],),
                device_id_type=pl.DeviceIdType.MESH,
            )

        for k in range(4):
            compute_chunk(4 * q + k)
        rdmas = []
        for d in (0, 1):
            sbB[0, d] = part_q(q, d)
            rd = b_rdma(0, 0, d)
            rd.start()
            rdmas.append(rd)
        for h in (1, 2, 3):
            if h == 1:
                for Qoff in (3, 1):
                    for k in range(4):
                        compute_chunk(4 * lax.rem(q + Qoff, 4) + k)
            elif h == 2:
                for k in range(4):
                    compute_chunk(4 * lax.rem(q + 2, 4) + k)
            for rd in rdmas:
                rd.wait()
            rdmas = []
            if h <= 2:
                slot = h % 2
                for d in (0, 1):
                    Q = lax.rem(q + (h if d else -h) + 4, 4)
                    sbB[slot, d] = part_q(Q, d) + rbB[d, h - 1]
                    rd = b_rdma(h, slot, d)
                    rd.start()
                    rdmas.append(rd)
        for d in (0, 1):
            Qo = lax.rem(q + (3 if d else 1), 4)
            accB[d] = part_q(Qo, d) + rbB[d, 2]

        def a_rdma(h, slot, d):
            return pltpu.make_async_remote_copy(
                src_ref=sbA.at[slot, d],
                dst_ref=rbA.at[d, h],
                send_sem=semA_send.at[slot, d],
                recv_sem=semA_recv.at[d, h],
                device_id=(col_tgt[d],),
                device_id_type=pl.DeviceIdType.MESH,
            )

        rdmas = []
        for d in (0, 1):
            sbA[0, d] = accB_z(zi, d)
            rd = a_rdma(0, 0, d)
            rd.start()
            rdmas.append(rd)
        for h in (1, 2, 3):
            for rd in rdmas:
                rd.wait()
            rdmas = []
            if h <= 2:
                slot = h % 2
                for d in (0, 1):
                    Z = lax.rem(zi + (h if d else -h) + 4, 4)
                    sbA[slot, d] = accB_z(Z, d) + rbA[d, h - 1]
                    rd = a_rdma(h, slot, d)
                    rd.start()
                    rdmas.append(rd)
        for d in (0, 1):
            Zo = lax.rem(zi + (3 if d else 1), 4)
            Qo = lax.rem(q + (3 if d else 1), 4)
            blk = accB_z(Zo, d) + rbA[d, 2]
            ownA[d] = blk.astype(bf)
            qbuf[d, pl.ds(Zo * CHUNK, CHUNK), :] = blk.astype(bf)
            store_chunk(Qo * 4 + Zo, d, blk)

        def c_rdma(h, d):
            src = ownA.at[d] if h == 0 else rbC.at[d, h - 1]
            return pltpu.make_async_remote_copy(
                src_ref=src,
                dst_ref=rbC.at[d, h],
                send_sem=semC_send.at[h % 2, d],
                recv_sem=semC_recv.at[d, h],
                device_id=(col_tgt[d],),
                device_id_type=pl.DeviceIdType.MESH,
            )

        rdmas = [c_rdma(0, 0), c_rdma(0, 1)]
        for rd in rdmas:
            rd.start()
        for h in (0, 1, 2):
            for rd in rdmas:
                rd.wait()
            if h < 2:
                rdmas = [c_rdma(h + 1, 0), c_rdma(h + 1, 1)]
                for rd in rdmas:
                    rd.start()
            for d in (0, 1):
                Zr = lax.rem(zi + (h if d else -h) + 4, 4)
                Qo = lax.rem(q + (3 if d else 1), 4)
                data = rbC[d, h]
                qbuf[d, pl.ds(Zr * CHUNK, CHUNK), :] = data
                store_chunk(Qo * 4 + Zr, d, data.astype(jnp.float32))

        def d_rdma(h, d):
            src = qbuf.at[d] if h == 0 else rbD.at[d, h - 1]
            return pltpu.make_async_remote_copy(
                src_ref=src,
                dst_ref=rbD.at[d, h],
                send_sem=semD_send.at[h % 2, d],
                recv_sem=semD_recv.at[d, h],
                device_id=(plane_tgt[d],),
                device_id_type=pl.DeviceIdType.MESH,
            )

        rdmas = [d_rdma(0, 0), d_rdma(0, 1)]
        for rd in rdmas:
            rd.start()
        for h in (0, 1, 2):
            for rd in rdmas:
                rd.wait()
            if h < 2:
                rdmas = [d_rdma(h + 1, 0), d_rdma(h + 1, 1)]
                for rd in rdmas:
                    rd.start()
            for d in (0, 1):
                Qr = lax.rem(q + (h if d else -h) + 4, 4)
                for k in range(4):
                    store_chunk(
                        Qr * 4 + k,
                        d,
                        rbD[d, h, pl.ds(k * CHUNK, CHUNK), :].astype(jnp.float32),
                    )

        @functools.partial(pl.run_scoped, sem=pltpu.SemaphoreType.REGULAR)
        def _(sem):
            for nbr in (ipn, ipp, cn, cp):
                pl.semaphore_signal(
                    sem, inc=1, device_id=(nbr,),
                    device_id_type=pl.DeviceIdType.MESH,
                )
            pl.semaphore_wait(sem, 4)

    return pl.pallas_call(
        body,
        out_shape=jax.ShapeDtypeStruct((SQ, D_MODEL), jnp.float32),
        in_specs=[pl.BlockSpec(memory_space=pltpu.VMEM)] * 5,
        out_specs=pl.BlockSpec(memory_space=pltpu.VMEM),
        scratch_shapes=[
            pltpu.VMEM((SQ, D_MODEL), jnp.float32),
            pltpu.VMEM((2, 2, QROWS, HALF), jnp.float32),
            pltpu.VMEM((2, 3, QROWS, HALF), jnp.float32),
            pltpu.VMEM((2, QROWS, HALF), jnp.float32),
            pltpu.VMEM((2, 2, CHUNK, HALF), jnp.float32),
            pltpu.VMEM((2, 3, CHUNK, HALF), jnp.float32),
            pltpu.VMEM((2, CHUNK, HALF), jnp.bfloat16),
            pltpu.VMEM((2, QROWS, HALF), jnp.bfloat16),
            pltpu.VMEM((2, 3, CHUNK, HALF), jnp.bfloat16),
            pltpu.VMEM((2, 3, QROWS, HALF), jnp.bfloat16),
            pltpu.SemaphoreType.DMA((2, 2)),
            pltpu.SemaphoreType.DMA((2, 3)),
            pltpu.SemaphoreType.DMA((2, 2)),
            pltpu.SemaphoreType.DMA((2, 3)),
            pltpu.SemaphoreType.DMA((2, 2)),
            pltpu.SemaphoreType.DMA((2, 3)),
            pltpu.SemaphoreType.DMA((2, 2)),
            pltpu.SemaphoreType.DMA((2, 3)),
        ],
        compiler_params=pltpu.CompilerParams(
            collective_id=0, vmem_limit_bytes=100 * 1024 * 1024
        ),
    )(xg, Wq_b, KgT, Vgt, Wo_r)


def kernel(x, Wq, K_ext, V_ext, Wo):
    out = _fused(*_prep(x, Wq, K_ext, V_ext, Wo))
    return out.reshape(1, SQ, D_MODEL)


# device time: 162759 ns/iter; 1.9379x vs baseline; 1.2294x over previous
import functools

import jax
import jax.numpy as jnp
from jax import lax
from jax.experimental import pallas as pl
from jax.experimental.pallas import tpu as pltpu

N_DEV = 16
SQ = 2048
D_MODEL = 1024
H_TOTAL = 128
H_PER = 8
DH = 128
CHUNK = SQ // N_DEV
QROWS = SQ // 4
HALF = D_MODEL // 2
SCALE = 0.08838834764831843


def _group_rows(t):
    return (
        t.reshape(8, 4, 64, *t.shape[1:])
        .transpose(1, 0, 2, *range(3, t.ndim + 2))
        .reshape(4 * 512, *t.shape[1:])
    )


def _prep(x, Wq, K_ext, V_ext, Wo):
    me = lax.axis_index("i")
    bf = jnp.bfloat16
    xg = _group_rows(x.reshape(SQ, D_MODEL)).astype(bf)
    K = lax.dynamic_slice_in_dim(
        K_ext.reshape(SQ, H_TOTAL, DH), me * H_PER, H_PER, axis=1
    )
    V = lax.dynamic_slice_in_dim(
        V_ext.reshape(SQ, H_TOTAL, DH), me * H_PER, H_PER, axis=1
    )
    Kg = _group_rows(K).reshape(4, 512, H_PER, DH)
    Vg = _group_rows(V).reshape(4, 512, H_PER, DH)
    KgT = Kg.transpose(0, 2, 3, 1).astype(bf)
    Vgt = Vg.transpose(0, 2, 1, 3).astype(bf)
    Wq_b = Wq.astype(bf)
    Wo_r = Wo.reshape(H_PER, DH, D_MODEL).astype(bf)
    return xg, Wq_b, KgT, Vgt, Wo_r


def _fused(xg, Wq_b, KgT, Vgt, Wo_r):
    def body(
        xg_ref,
        wq_ref,
        kgt_ref,
        vgt_ref,
        wo_ref,
        out_ref,
        part_ref,
        sbB,
        rbB,
        accB,
        sbA,
        rbA,
        ownA,
        qbuf,
        rbC,
        rbD,
        semB_send,
        semB_recv,
        semA_send,
        semA_recv,
        semC_send,
        semC_recv,
        semD_send,
        semD_recv,
    ):
        me = lax.axis_index("i")
        bf = jnp.bfloat16
        q = lax.rem(me, 4)
        zi = me // 4
        pbase = me - q
        ipn = pbase + lax.rem(q + 1, 4)
        ipp = pbase + lax.rem(q + 3, 4)
        cn = lax.rem(zi + 1, 4) * 4 + q
        cp = lax.rem(zi + 3, 4) * 4 + q
        plane_tgt = (ipn, ipp)
        col_tgt = (cn, cp)

        barrier = pltpu.get_barrier_semaphore()
        for nbr in (ipn, ipp, cn, cp):
            pl.semaphore_signal(
                barrier, inc=1, device_id=(nbr,),
                device_id_type=pl.DeviceIdType.MESH,
            )
        pl.semaphore_wait(barrier, 4)

        def compute_quarter(Q):
            xq = xg_ref[pl.ds(Q * QROWS, QROWS), :]
            qc = jnp.dot(xq, wq_ref[:, :], preferred_element_type=jnp.float32)
            qc = (qc * SCALE).astype(bf)
            acc = jnp.zeros((QROWS, D_MODEL), jnp.float32)
            for h in range(H_PER):
                qh = qc[:, h * DH : (h + 1) * DH]
                kh = kgt_ref[Q, h]
                s = jnp.dot(qh, kh, preferred_element_type=jnp.float32)
                e = jnp.exp(s)
                w = (e / jnp.sum(e, axis=-1, keepdims=True)).astype(bf)
                ctx = jnp.dot(
                    w, vgt_ref[Q, h], preferred_element_type=jnp.float32
                ).astype(bf)
                acc = acc + jnp.dot(
                    ctx, wo_ref[h], preferred_element_type=jnp.float32
                )
            part_ref[pl.ds(Q * QROWS, QROWS), :] = acc

        def store_chunk(c, d, data):
            j = lax.rem(c, 4)
            r = c // 4
            b0 = 8 * j + r
            col = pl.ds(d * HALF, HALF)
            out_ref[pl.ds(b0 * 64, 64), col] = data[:64]
            out_ref[pl.ds(b0 * 64 + 256, 64), col] = data[64:]

        def part_q(Q, d):
            return part_ref[pl.ds(Q * QROWS, QROWS), pl.ds(d * HALF, HALF)]

        def accB_z(Z, d):
            return accB[d, pl.ds(Z * CHUNK, CHUNK), :]

        def b_rdma(h, slot, d):
            return pltpu.make_async_remote_copy(
                src_ref=sbB.at[slot, d],
                dst_ref=rbB.at[d, h],
                send_sem=semB_send.at[slot, d],
                recv_sem=semB_recv.at[d, h],
                device_id=(plane_tgt[d],),
                device_id_type=pl.DeviceIdType.MESH,
            )

        compute_quarter(q)
        rdmas = []
        for d in (0, 1):
            sbB[0, d] = part_q(q, d)
            rd = b_rdma(0, 0, d)
            rd.start()
            rdmas.append(rd)
        for h in (1, 2, 3):
            if h == 1:
                compute_quarter(lax.rem(q + 3, 4))
                compute_quarter(lax.rem(q + 1, 4))
            elif h == 2:
                compute_quarter(lax.rem(q + 2, 4))
            for rd in rdmas:
                rd.wait()
            rdmas = []
            if h <= 2:
                slot = h % 2
                for d in (0, 1):
                    Q = lax.rem(q + (h if d else -h) + 4, 4)
                    sbB[slot, d] = part_q(Q, d) + rbB[d, h - 1]
                    rd = b_rdma(h, slot, d)
                    rd.start()
                    rdmas.append(rd)
        for d in (0, 1):
            Qo = lax.rem(q + (3 if d else 1), 4)
            accB[d] = part_q(Qo, d) + rbB[d, 2]

        def a_rdma(h, slot, d):
            return pltpu.make_async_remote_copy(
                src_ref=sbA.at[slot, d],
                dst_ref=rbA.at[d, h],
                send_sem=semA_send.at[slot, d],
                recv_sem=semA_recv.at[d, h],
                device_id=(col_tgt[d],),
                device_id_type=pl.DeviceIdType.MESH,
            )

        rdmas = []
        for d in (0, 1):
            sbA[0, d] = accB_z(zi, d)
            rd = a_rdma(0, 0, d)
            rd.start()
            rdmas.append(rd)
        for h in (1, 2, 3):
            for rd in rdmas:
                rd.wait()
            rdmas = []
            if h <= 2:
                slot = h % 2
                for d in (0, 1):
                    Z = lax.rem(zi + (h if d else -h) + 4, 4)
                    sbA[slot, d] = accB_z(Z, d) + rbA[d, h - 1]
                    rd = a_rdma(h, slot, d)
                    rd.start()
                    rdmas.append(rd)
        for d in (0, 1):
            Zo = lax.rem(zi + (3 if d else 1), 4)
            Qo = lax.rem(q + (3 if d else 1), 4)
            blk = accB_z(Zo, d) + rbA[d, 2]
            ownA[d] = blk.astype(bf)
            qbuf[d, pl.ds(Zo * CHUNK, CHUNK), :] = blk.astype(bf)
            store_chunk(Qo * 4 + Zo, d, blk)

        def c_rdma(h, d):
            src = ownA.at[d] if h == 0 else rbC.at[d, h - 1]
            return pltpu.make_async_remote_copy(
                src_ref=src,
                dst_ref=rbC.at[d, h],
                send_sem=semC_send.at[h % 2, d],
                recv_sem=semC_recv.at[d, h],
                device_id=(col_tgt[d],),
                device_id_type=pl.DeviceIdType.MESH,
            )

        rdmas = [c_rdma(0, 0), c_rdma(0, 1)]
        for rd in rdmas:
            rd.start()
        for h in (0, 1, 2):
            for rd in rdmas:
                rd.wait()
            if h < 2:
                rdmas = [c_rdma(h + 1, 0), c_rdma(h + 1, 1)]
                for rd in rdmas:
                    rd.start()
            for d in (0, 1):
                Zr = lax.rem(zi + (h if d else -h) + 4, 4)
                Qo = lax.rem(q + (3 if d else 1), 4)
                data = rbC[d, h]
                qbuf[d, pl.ds(Zr * CHUNK, CHUNK), :] = data
                store_chunk(Qo * 4 + Zr, d, data.astype(jnp.float32))

        def d_rdma(h, d):
            src = qbuf.at[d] if h == 0 else rbD.at[d, h - 1]
            return pltpu.make_async_remote_copy(
                src_ref=src,
                dst_ref=rbD.at[d, h],
                send_sem=semD_send.at[h % 2, d],
                recv_sem=semD_recv.at[d, h],
                device_id=(plane_tgt[d],),
                device_id_type=pl.DeviceIdType.MESH,
            )

        rdmas = [d_rdma(0, 0), d_rdma(0, 1)]
        for rd in rdmas:
            rd.start()
        for h in (0, 1, 2):
            for rd in rdmas:
                rd.wait()
            if h < 2:
                rdmas = [d_rdma(h + 1, 0), d_rdma(h + 1, 1)]
                for rd in rdmas:
                    rd.start()
            for d in (0, 1):
                Qr = lax.rem(q + (h if d else -h) + 4, 4)
                for k in range(4):
                    store_chunk(
                        Qr * 4 + k,
                        d,
                        rbD[d, h, pl.ds(k * CHUNK, CHUNK), :].astype(jnp.float32),
                    )

        @functools.partial(pl.run_scoped, sem=pltpu.SemaphoreType.REGULAR)
        def _(sem):
            for nbr in (ipn, ipp, cn, cp):
                pl.semaphore_signal(
                    sem, inc=1, device_id=(nbr,),
                    device_id_type=pl.DeviceIdType.MESH,
                )
            pl.semaphore_wait(sem, 4)

    return pl.pallas_call(
        body,
        out_shape=jax.ShapeDtypeStruct((SQ, D_MODEL), jnp.float32),
        in_specs=[pl.BlockSpec(memory_space=pltpu.VMEM)] * 5,
        out_specs=pl.BlockSpec(memory_space=pltpu.VMEM),
        scratch_shapes=[
            pltpu.VMEM((SQ, D_MODEL), jnp.float32),
            pltpu.VMEM((2, 2, QROWS, HALF), jnp.float32),
            pltpu.VMEM((2, 3, QROWS, HALF), jnp.float32),
            pltpu.VMEM((2, QROWS, HALF), jnp.float32),
            pltpu.VMEM((2, 2, CHUNK, HALF), jnp.float32),
            pltpu.VMEM((2, 3, CHUNK, HALF), jnp.float32),
            pltpu.VMEM((2, CHUNK, HALF), jnp.bfloat16),
            pltpu.VMEM((2, QROWS, HALF), jnp.bfloat16),
            pltpu.VMEM((2, 3, CHUNK, HALF), jnp.bfloat16),
            pltpu.VMEM((2, 3, QROWS, HALF), jnp.bfloat16),
            pltpu.SemaphoreType.DMA((2, 2)),
            pltpu.SemaphoreType.DMA((2, 3)),
            pltpu.SemaphoreType.DMA((2, 2)),
            pltpu.SemaphoreType.DMA((2, 3)),
            pltpu.SemaphoreType.DMA((2, 2)),
            pltpu.SemaphoreType.DMA((2, 3)),
            pltpu.SemaphoreType.DMA((2, 2)),
            pltpu.SemaphoreType.DMA((2, 3)),
        ],
        compiler_params=pltpu.CompilerParams(
            collective_id=0, vmem_limit_bytes=100 * 1024 * 1024
        ),
    )(xg, Wq_b, KgT, Vgt, Wo_r)


def kernel(x, Wq, K_ext, V_ext, Wo):
    out = _fused(*_prep(x, Wq, K_ext, V_ext, Wo))
    return out.reshape(1, SQ, D_MODEL)


# device time: 144352 ns/iter; 2.1850x vs baseline; 1.1275x over previous
import functools

import jax
import jax.numpy as jnp
from jax import lax
from jax.experimental import pallas as pl
from jax.experimental.pallas import tpu as pltpu

N_DEV = 16
SQ = 2048
D_MODEL = 1024
H_TOTAL = 128
H_PER = 8
DH = 128
CHUNK = SQ // N_DEV
QROWS = SQ // 4
HALF = D_MODEL // 2
SCALE = 0.08838834764831843


def _group_rows(t):
    return (
        t.reshape(8, 4, 64, *t.shape[1:])
        .transpose(1, 0, 2, *range(3, t.ndim + 2))
        .reshape(4 * 512, *t.shape[1:])
    )


def _prep(x, Wq, K_ext, V_ext, Wo):
    me = lax.axis_index("i")
    bf = jnp.bfloat16
    xg = _group_rows(x.reshape(SQ, D_MODEL)).astype(bf)
    K = lax.dynamic_slice_in_dim(
        K_ext.reshape(SQ, H_TOTAL, DH), me * H_PER, H_PER, axis=1
    )
    V = lax.dynamic_slice_in_dim(
        V_ext.reshape(SQ, H_TOTAL, DH), me * H_PER, H_PER, axis=1
    )
    Kg = _group_rows(K).reshape(4, 512, H_PER, DH)
    Vg = _group_rows(V).reshape(4, 512, H_PER, DH)
    KgT = Kg.transpose(0, 2, 3, 1).astype(bf)
    Vgt = Vg.transpose(0, 2, 1, 3).astype(bf)
    Wq_b = Wq.astype(bf)
    Wo_r = Wo.reshape(H_PER, DH, D_MODEL).astype(bf)
    return xg, Wq_b, KgT, Vgt, Wo_r


def _fused(xg, Wq_b, KgT, Vgt, Wo_r):
    def body(
        xg_ref,
        wq_ref,
        kgt_ref,
        vgt_ref,
        wo_ref,
        out_ref,
        part_ref,
        sbB,
        rbB,
        accB,
        sbA,
        rbA,
        ownA,
        qbuf,
        rbC,
        rbD,
        semB_send,
        semB_recv,
        semA_send,
        semA_recv,
        semC_send,
        semC_recv,
        semD_send,
        semD_recv,
    ):
        me = lax.axis_index("i")
        bf = jnp.bfloat16
        q = lax.rem(me, 4)
        zi = me // 4
        pbase = me - q
        ipn = pbase + lax.rem(q + 1, 4)
        ipp = pbase + lax.rem(q + 3, 4)
        cn = lax.rem(zi + 1, 4) * 4 + q
        cp = lax.rem(zi + 3, 4) * 4 + q
        plane_tgt = (ipn, ipp)
        col_tgt = (cn, cp)

        barrier = pltpu.get_barrier_semaphore()
        for nbr in (ipn, ipp, cn, cp):
            pl.semaphore_signal(
                barrier, inc=1, device_id=(nbr,),
                device_id_type=pl.DeviceIdType.MESH,
            )
        pl.semaphore_wait(barrier, 4)

        def compute_quarter(Q):
            xq = xg_ref[pl.ds(Q * QROWS, QROWS), :]
            qc = jnp.dot(xq, wq_ref[:, :], preferred_element_type=jnp.float32)
            qc = (qc * SCALE).astype(bf)
            acc = jnp.zeros((QROWS, D_MODEL), jnp.float32)
            for h in range(H_PER):
                qh = qc[:, h * DH : (h + 1) * DH]
                kh = kgt_ref[Q, h]
                s = jnp.dot(qh, kh, preferred_element_type=jnp.float32)
                e = jnp.exp(s)
                w = (e / jnp.sum(e, axis=-1, keepdims=True)).astype(bf)
                ctx = jnp.dot(
                    w, vgt_ref[Q, h], preferred_element_type=jnp.float32
                ).astype(bf)
                acc = acc + jnp.dot(
                    ctx, wo_ref[h], preferred_element_type=jnp.float32
                )
            part_ref[pl.ds(Q * QROWS, QROWS), :] = acc

        def store_chunk(c, d, data):
            j = lax.rem(c, 4)
            r = c // 4
            b0 = 8 * j + r
            col = pl.ds(d * HALF, HALF)
            out_ref[pl.ds(b0 * 64, 64), col] = data[:64]
            out_ref[pl.ds(b0 * 64 + 256, 64), col] = data[64:]

        def part_q(Q, d):
            return part_ref[pl.ds(Q * QROWS, QROWS), pl.ds(d * HALF, HALF)]

        def accB_z(Z, d):
            return accB[d, pl.ds(Z * CHUNK, CHUNK), :]

        def b_rdma(h, slot, d):
            return pltpu.make_async_remote_copy(
                src_ref=sbB.at[slot, d],
                dst_ref=rbB.at[d, h],
                send_sem=semB_send.at[slot, d],
                recv_sem=semB_recv.at[d, h],
                device_id=(plane_tgt[d],),
                device_id_type=pl.DeviceIdType.MESH,
            )

        compute_quarter(q)
        rdmas = []
        for d in (0, 1):
            sbB[0, d] = part_q(q, d).astype(bf)
            rd = b_rdma(0, 0, d)
            rd.start()
            rdmas.append(rd)
        for h in (1, 2, 3):
            if h == 1:
                compute_quarter(lax.rem(q + 3, 4))
                compute_quarter(lax.rem(q + 1, 4))
            elif h == 2:
                compute_quarter(lax.rem(q + 2, 4))
            for rd in rdmas:
                rd.wait()
            rdmas = []
            if h <= 2:
                slot = h % 2
                for d in (0, 1):
                    Q = lax.rem(q + (h if d else -h) + 4, 4)
                    sbB[slot, d] = (part_q(Q, d) + rbB[d, h - 1]).astype(bf)
                    rd = b_rdma(h, slot, d)
                    rd.start()
                    rdmas.append(rd)
        for d in (0, 1):
            Qo = lax.rem(q + (3 if d else 1), 4)
            accB[d] = part_q(Qo, d) + rbB[d, 2]

        def a_rdma(h, slot, d):
            return pltpu.make_async_remote_copy(
                src_ref=sbA.at[slot, d],
                dst_ref=rbA.at[d, h],
                send_sem=semA_send.at[slot, d],
                recv_sem=semA_recv.at[d, h],
                device_id=(col_tgt[d],),
                device_id_type=pl.DeviceIdType.MESH,
            )

        rdmas = []
        for d in (0, 1):
            sbA[0, d] = accB_z(zi, d).astype(bf)
            rd = a_rdma(0, 0, d)
            rd.start()
            rdmas.append(rd)
        for h in (1, 2, 3):
            for rd in rdmas:
                rd.wait()
            rdmas = []
            if h <= 2:
                slot = h % 2
                for d in (0, 1):
                    Z = lax.rem(zi + (h if d else -h) + 4, 4)
                    sbA[slot, d] = (accB_z(Z, d) + rbA[d, h - 1]).astype(bf)
                    rd = a_rdma(h, slot, d)
                    rd.start()
                    rdmas.append(rd)
        for d in (0, 1):
            Zo = lax.rem(zi + (3 if d else 1), 4)
            Qo = lax.rem(q + (3 if d else 1), 4)
            blk = accB_z(Zo, d) + rbA[d, 2]
            ownA[d] = blk.astype(bf)
            qbuf[d, pl.ds(Zo * CHUNK, CHUNK), :] = blk.astype(bf)
            store_chunk(Qo * 4 + Zo, d, blk)

        def c_rdma(h, d):
            src = ownA.at[d] if h == 0 else rbC.at[d, h - 1]
            return pltpu.make_async_remote_copy(
                src_ref=src,
                dst_ref=rbC.at[d, h],
                send_sem=semC_send.at[h % 2, d],
                recv_sem=semC_recv.at[d, h],
                device_id=(col_tgt[d],),
                device_id_type=pl.DeviceIdType.MESH,
            )

        rdmas = [c_rdma(0, 0), c_rdma(0, 1)]
        for rd in rdmas:
            rd.start()
        for h in (0, 1, 2):
            for rd in rdmas:
                rd.wait()
            if h < 2:
                rdmas = [c_rdma(h + 1, 0), c_rdma(h + 1, 1)]
                for rd in rdmas:
                    rd.start()
            for d in (0, 1):
                Zr = lax.rem(zi + (h if d else -h) + 4, 4)
                Qo = lax.rem(q + (3 if d else 1), 4)
                data = rbC[d, h]
                qbuf[d, pl.ds(Zr * CHUNK, CHUNK), :] = data
                store_chunk(Qo * 4 + Zr, d, data.astype(jnp.float32))

        def d_rdma(h, d):
            src = qbuf.at[d] if h == 0 else rbD.at[d, h - 1]
            return pltpu.make_async_remote_copy(
                src_ref=src,
                dst_ref=rbD.at[d, h],
                send_sem=semD_send.at[h % 2, d],
                recv_sem=semD_recv.at[d, h],
                device_id=(plane_tgt[d],),
                device_id_type=pl.DeviceIdType.MESH,
            )

        rdmas = [d_rdma(0, 0), d_rdma(0, 1)]
        for rd in rdmas:
            rd.start()
        for h in (0, 1, 2):
            for rd in rdmas:
                rd.wait()
            if h < 2:
                rdmas = [d_rdma(h + 1, 0), d_rdma(h + 1, 1)]
                for rd in rdmas:
                    rd.start()
            for d in (0, 1):
                Qr = lax.rem(q + (h if d else -h) + 4, 4)
                for k in range(4):
                    store_chunk(
                        Qr * 4 + k,
                        d,
                        rbD[d, h, pl.ds(k * CHUNK, CHUNK), :].astype(jnp.float32),
                    )

        @functools.partial(pl.run_scoped, sem=pltpu.SemaphoreType.REGULAR)
        def _(sem):
            for nbr in (ipn, ipp, cn, cp):
                pl.semaphore_signal(
                    sem, inc=1, device_id=(nbr,),
                    device_id_type=pl.DeviceIdType.MESH,
                )
            pl.semaphore_wait(sem, 4)

    return pl.pallas_call(
        body,
        out_shape=jax.ShapeDtypeStruct((SQ, D_MODEL), jnp.float32),
        in_specs=[pl.BlockSpec(memory_space=pltpu.VMEM)] * 5,
        out_specs=pl.BlockSpec(memory_space=pltpu.VMEM),
        scratch_shapes=[
            pltpu.VMEM((SQ, D_MODEL), jnp.float32),
            pltpu.VMEM((2, 2, QROWS, HALF), jnp.bfloat16),
            pltpu.VMEM((2, 3, QROWS, HALF), jnp.bfloat16),
            pltpu.VMEM((2, QROWS, HALF), jnp.float32),
            pltpu.VMEM((2, 2, CHUNK, HALF), jnp.bfloat16),
            pltpu.VMEM((2, 3, CHUNK, HALF), jnp.bfloat16),
            pltpu.VMEM((2, CHUNK, HALF), jnp.bfloat16),
            pltpu.VMEM((2, QROWS, HALF), jnp.bfloat16),
            pltpu.VMEM((2, 3, CHUNK, HALF), jnp.bfloat16),
            pltpu.VMEM((2, 3, QROWS, HALF), jnp.bfloat16),
            pltpu.SemaphoreType.DMA((2, 2)),
            pltpu.SemaphoreType.DMA((2, 3)),
            pltpu.SemaphoreType.DMA((2, 2)),
            pltpu.SemaphoreType.DMA((2, 3)),
            pltpu.SemaphoreType.DMA((2, 2)),
            pltpu.SemaphoreType.DMA((2, 3)),
            pltpu.SemaphoreType.DMA((2, 2)),
            pltpu.SemaphoreType.DMA((2, 3)),
        ],
        compiler_params=pltpu.CompilerParams(
            collective_id=0, vmem_limit_bytes=100 * 1024 * 1024
        ),
    )(xg, Wq_b, KgT, Vgt, Wo_r)


def kernel(x, Wq, K_ext, V_ext, Wo):
    out = _fused(*_prep(x, Wq, K_ext, V_ext, Wo))
    return out.reshape(1, SQ, D_MODEL)


# device time: 141263 ns/iter; 2.2328x vs baseline; 1.0219x over previous
import functools

import jax
import jax.numpy as jnp
from jax import lax
from jax.experimental import pallas as pl
from jax.experimental.pallas import tpu as pltpu

N_DEV = 16
SQ = 2048
D_MODEL = 1024
H_TOTAL = 128
H_PER = 8
DH = 128
CHUNK = SQ // N_DEV
QROWS = SQ // 4
HALF = D_MODEL // 2
SCALE = 0.08838834764831843


def _prep(x, Wq, K_ext, V_ext, Wo):
    me = lax.axis_index("i")
    bf = jnp.bfloat16
    xb = x.reshape(SQ, D_MODEL).astype(bf)
    K = lax.dynamic_slice_in_dim(
        K_ext.reshape(SQ, H_TOTAL, DH), me * H_PER, H_PER, axis=1
    )
    V = lax.dynamic_slice_in_dim(
        V_ext.reshape(SQ, H_TOTAL, DH), me * H_PER, H_PER, axis=1
    )
    Kb = K.reshape(SQ, H_PER * DH).astype(bf)
    Vb = V.reshape(SQ, H_PER * DH).astype(bf)
    Wq_b = Wq.astype(bf)
    Wo_r = Wo.reshape(H_PER, DH, D_MODEL).astype(bf)
    return xb, Wq_b, Kb, Vb, Wo_r


def _fused(xb, Wq_b, Kb, Vb, Wo_r):
    def body(
        xb_ref,
        wq_ref,
        kb_ref,
        vb_ref,
        wo_ref,
        out_ref,
        part_ref,
        sbB,
        rbB,
        accB,
        sbA,
        rbA,
        ownA,
        qbuf,
        rbC,
        rbD,
        semB_send,
        semB_recv,
        semA_send,
        semA_recv,
        semC_send,
        semC_recv,
        semD_send,
        semD_recv,
    ):
        me = lax.axis_index("i")
        bf = jnp.bfloat16
        q = lax.rem(me, 4)
        zi = me // 4
        pbase = me - q
        ipn = pbase + lax.rem(q + 1, 4)
        ipp = pbase + lax.rem(q + 3, 4)
        cn = lax.rem(zi + 1, 4) * 4 + q
        cp = lax.rem(zi + 3, 4) * 4 + q
        plane_tgt = (ipn, ipp)
        col_tgt = (cn, cp)

        barrier = pltpu.get_barrier_semaphore()
        for nbr in (ipn, ipp, cn, cp):
            pl.semaphore_signal(
                barrier, inc=1, device_id=(nbr,),
                device_id_type=pl.DeviceIdType.MESH,
            )
        pl.semaphore_wait(barrier, 4)

        def gather_phase(ref, Q):
            return jnp.concatenate(
                [ref[pl.ds(g * 256 + Q * 64, 64), :] for g in range(8)], axis=0
            )

        def compute_quarter(Q):
            xq = gather_phase(xb_ref, Q)
            kq = gather_phase(kb_ref, Q)
            vq = gather_phase(vb_ref, Q)
            qc = jnp.dot(xq, wq_ref[:, :], preferred_element_type=jnp.float32)
            qc = (qc * SCALE).astype(bf)
            acc = jnp.zeros((QROWS, D_MODEL), jnp.float32)
            for h in range(H_PER):
                cols = slice(h * DH, (h + 1) * DH)
                s = lax.dot_general(
                    qc[:, cols],
                    kq[:, cols],
                    (((1,), (1,)), ((), ())),
                    preferred_element_type=jnp.float32,
                )
                e = jnp.exp(s)
                w = (e / jnp.sum(e, axis=-1, keepdims=True)).astype(bf)
                ctx = jnp.dot(
                    w, vq[:, cols], preferred_element_type=jnp.float32
                ).astype(bf)
                acc = acc + jnp.dot(
                    ctx, wo_ref[h], preferred_element_type=jnp.float32
                )
            part_ref[pl.ds(Q * QROWS, QROWS), :] = acc

        def store_chunk(c, d, data):
            j = lax.rem(c, 4)
            r = c // 4
            b0 = 8 * j + r
            col = pl.ds(d * HALF, HALF)
            out_ref[pl.ds(b0 * 64, 64), col] = data[:64]
            out_ref[pl.ds(b0 * 64 + 256, 64), col] = data[64:]

        def part_q(Q, d):
            return part_ref[pl.ds(Q * QROWS, QROWS), pl.ds(d * HALF, HALF)]

        def accB_z(Z, d):
            return accB[d, pl.ds(Z * CHUNK, CHUNK), :]

        def b_rdma(h, slot, d):
            return pltpu.make_async_remote_copy(
                src_ref=sbB.at[slot, d],
                dst_ref=rbB.at[d, h],
                send_sem=semB_send.at[slot, d],
                recv_sem=semB_recv.at[d, h],
                device_id=(plane_tgt[d],),
                device_id_type=pl.DeviceIdType.MESH,
            )

        compute_quarter(q)
        rdmas = []
        for d in (0, 1):
            sbB[0, d] = part_q(q, d).astype(bf)
            rd = b_rdma(0, 0, d)
            rd.start()
            rdmas.append(rd)
        for h in (1, 2, 3):
            if h == 1:
                compute_quarter(lax.rem(q + 3, 4))
                compute_quarter(lax.rem(q + 1, 4))
            elif h == 2:
                compute_quarter(lax.rem(q + 2, 4))
            for rd in rdmas:
                rd.wait()
            rdmas = []
            if h <= 2:
                slot = h % 2
                for d in (0, 1):
                    Q = lax.rem(q + (h if d else -h) + 4, 4)
                    sbB[slot, d] = (part_q(Q, d) + rbB[d, h - 1]).astype(bf)
                    rd = b_rdma(h, slot, d)
                    rd.start()
                    rdmas.append(rd)
        for d in (0, 1):
            Qo = lax.rem(q + (3 if d else 1), 4)
            accB[d] = part_q(Qo, d) + rbB[d, 2]

        def a_rdma(h, slot, d):
            return pltpu.make_async_remote_copy(
                src_ref=sbA.at[slot, d],
                dst_ref=rbA.at[d, h],
                send_sem=semA_send.at[slot, d],
                recv_sem=semA_recv.at[d, h],
                device_id=(col_tgt[d],),
                device_id_type=pl.DeviceIdType.MESH,
            )

        rdmas = []
        for d in (0, 1):
            sbA[0, d] = accB_z(zi, d).astype(bf)
            rd = a_rdma(0, 0, d)
            rd.start()
            rdmas.append(rd)
        for h in (1, 2, 3):
            for rd in rdmas:
                rd.wait()
            rdmas = []
            if h <= 2:
                slot = h % 2
                for d in (0, 1):
                    Z = lax.rem(zi + (h if d else -h) + 4, 4)
                    sbA[slot, d] = (accB_z(Z, d) + rbA[d, h - 1]).astype(bf)
                    rd = a_rdma(h, slot, d)
                    rd.start()
                    rdmas.append(rd)
        for d in (0, 1):
            Zo = lax.rem(zi + (3 if d else 1), 4)
            Qo = lax.rem(q + (3 if d else 1), 4)
            blk = accB_z(Zo, d) + rbA[d, 2]
            ownA[d] = blk.astype(bf)
            qbuf[d, pl.ds(Zo * CHUNK, CHUNK), :] = blk.astype(bf)
            store_chunk(Qo * 4 + Zo, d, blk)

        def c_rdma(h, d):
            src = ownA.at[d] if h == 0 else rbC.at[d, h - 1]
            return pltpu.make_async_remote_copy(
                src_ref=src,
                dst_ref=rbC.at[d, h],
                send_sem=semC_send.at[h % 2, d],
                recv_sem=semC_recv.at[d, h],
                device_id=(col_tgt[d],),
                device_id_type=pl.DeviceIdType.MESH,
            )

        rdmas = [c_rdma(0, 0), c_rdma(0, 1)]
        for rd in rdmas:
            rd.start()
        for h in (0, 1, 2):
            for rd in rdmas:
                rd.wait()
            if h < 2:
                rdmas = [c_rdma(h + 1, 0), c_rdma(h + 1, 1)]
                for rd in rdmas:
                    rd.start()
            for d in (0, 1):
                Zr = lax.rem(zi + (h if d else -h) + 4, 4)
                Qo = lax.rem(q + (3 if d else 1), 4)
                data = rbC[d, h]
                qbuf[d, pl.ds(Zr * CHUNK, CHUNK), :] = data
                store_chunk(Qo * 4 + Zr, d, data.astype(jnp.float32))

        def d_rdma(h, d):
            src = qbuf.at[d] if h == 0 else rbD.at[d, h - 1]
            return pltpu.make_async_remote_copy(
                src_ref=src,
                dst_ref=rbD.at[d, h],
                send_sem=semD_send.at[h % 2, d],
                recv_sem=semD_recv.at[d, h],
                device_id=(plane_tgt[d],),
                device_id_type=pl.DeviceIdType.MESH,
            )

        rdmas = [d_rdma(0, 0), d_rdma(0, 1)]
        for rd in rdmas:
            rd.start()
        for h in (0, 1, 2):
            for rd in rdmas:
                rd.wait()
            if h < 2:
                rdmas = [d_rdma(h + 1, 0), d_rdma(h + 1, 1)]
                for rd in rdmas:
                    rd.start()
            for d in (0, 1):
                Qr = lax.rem(q + (h if d else -h) + 4, 4)
                for k in range(4):
                    store_chunk(
                        Qr * 4 + k,
                        d,
                        rbD[d, h, pl.ds(k * CHUNK, CHUNK), :].astype(jnp.float32),
                    )

        @functools.partial(pl.run_scoped, sem=pltpu.SemaphoreType.REGULAR)
        def _(sem):
            for nbr in (ipn, ipp, cn, cp):
                pl.semaphore_signal(
                    sem, inc=1, device_id=(nbr,),
                    device_id_type=pl.DeviceIdType.MESH,
                )
            pl.semaphore_wait(sem, 4)

    return pl.pallas_call(
        body,
        out_shape=jax.ShapeDtypeStruct((SQ, D_MODEL), jnp.float32),
        in_specs=[pl.BlockSpec(memory_space=pltpu.VMEM)] * 5,
        out_specs=pl.BlockSpec(memory_space=pltpu.VMEM),
        scratch_shapes=[
            pltpu.VMEM((SQ, D_MODEL), jnp.float32),
            pltpu.VMEM((2, 2, QROWS, HALF), jnp.bfloat16),
            pltpu.VMEM((2, 3, QROWS, HALF), jnp.bfloat16),
            pltpu.VMEM((2, QROWS, HALF), jnp.float32),
            pltpu.VMEM((2, 2, CHUNK, HALF), jnp.bfloat16),
            pltpu.VMEM((2, 3, CHUNK, HALF), jnp.bfloat16),
            pltpu.VMEM((2, CHUNK, HALF), jnp.bfloat16),
            pltpu.VMEM((2, QROWS, HALF), jnp.bfloat16),
            pltpu.VMEM((2, 3, CHUNK, HALF), jnp.bfloat16),
            pltpu.VMEM((2, 3, QROWS, HALF), jnp.bfloat16),
            pltpu.SemaphoreType.DMA((2, 2)),
            pltpu.SemaphoreType.DMA((2, 3)),
            pltpu.SemaphoreType.DMA((2, 2)),
            pltpu.SemaphoreType.DMA((2, 3)),
            pltpu.SemaphoreType.DMA((2, 2)),
            pltpu.SemaphoreType.DMA((2, 3)),
            pltpu.SemaphoreType.DMA((2, 2)),
            pltpu.SemaphoreType.DMA((2, 3)),
        ],
        compiler_params=pltpu.CompilerParams(
            collective_id=0, vmem_limit_bytes=100 * 1024 * 1024
        ),
    )(xb, Wq_b, Kb, Vb, Wo_r)


def kernel(x, Wq, K_ext, V_ext, Wo):
    out = _fused(*_prep(x, Wq, K_ext, V_ext, Wo))
    return out.reshape(1, SQ, D_MODEL)


# device time: 136391 ns/iter; 2.3125x vs baseline; 1.0357x over previous
import functools

import jax
import jax.numpy as jnp
from jax import lax
from jax.experimental import pallas as pl
from jax.experimental.pallas import tpu as pltpu

N_DEV = 16
SQ = 2048
D_MODEL = 1024
H_TOTAL = 128
H_PER = 8
DH = 128
CHUNK = SQ // N_DEV
QROWS = SQ // 4
HALF = D_MODEL // 2
SCALE = 0.08838834764831843


def _prep(x, Wq, K_ext, V_ext, Wo):
    me = lax.axis_index("i")
    bf = jnp.bfloat16
    xb = x.reshape(SQ, D_MODEL).astype(bf)
    K = lax.dynamic_slice_in_dim(
        K_ext.reshape(SQ, H_TOTAL, DH), me * H_PER, H_PER, axis=1
    )
    V = lax.dynamic_slice_in_dim(
        V_ext.reshape(SQ, H_TOTAL, DH), me * H_PER, H_PER, axis=1
    )
    Kb = K.reshape(SQ, H_PER * DH).astype(bf)
    Vb = V.reshape(SQ, H_PER * DH).astype(bf)
    Wq_b = (Wq * SCALE).astype(bf)
    Wo_r = Wo.reshape(H_PER, DH, D_MODEL).astype(bf)
    return xb, Wq_b, Kb, Vb, Wo_r


def _fused(xb, Wq_b, Kb, Vb, Wo_r):
    def body(
        xb_ref,
        wq_ref,
        kb_ref,
        vb_ref,
        wo_ref,
        out_ref,
        part_ref,
        sbB,
        rbB,
        accB,
        sbA,
        rbA,
        ownA,
        qbuf,
        rbC,
        rbD,
        semB_send,
        semB_recv,
        semA_send,
        semA_recv,
        semC_send,
        semC_recv,
        semD_send,
        semD_recv,
    ):
        me = lax.axis_index("i")
        bf = jnp.bfloat16
        q = lax.rem(me, 4)
        zi = me // 4
        pbase = me - q
        ipn = pbase + lax.rem(q + 1, 4)
        ipp = pbase + lax.rem(q + 3, 4)
        cn = lax.rem(zi + 1, 4) * 4 + q
        cp = lax.rem(zi + 3, 4) * 4 + q
        plane_tgt = (ipn, ipp)
        col_tgt = (cn, cp)

        barrier = pltpu.get_barrier_semaphore()
        for nbr in (ipn, ipp, cn, cp):
            pl.semaphore_signal(
                barrier, inc=1, device_id=(nbr,),
                device_id_type=pl.DeviceIdType.MESH,
            )
        pl.semaphore_wait(barrier, 4)

        def gather_phase(ref, Q):
            return jnp.concatenate(
                [ref[pl.ds(g * 256 + Q * 64, 64), :] for g in range(8)], axis=0
            )

        def compute_quarter(Q):
            xq = gather_phase(xb_ref, Q)
            kq = gather_phase(kb_ref, Q)
            vq = gather_phase(vb_ref, Q)
            qc = jnp.dot(
                xq, wq_ref[:, :], preferred_element_type=jnp.float32
            ).astype(bf)
            acc = jnp.zeros((QROWS, D_MODEL), jnp.float32)
            for h in range(H_PER):
                cols = slice(h * DH, (h + 1) * DH)
                s = lax.dot_general(
                    qc[:, cols],
                    kq[:, cols],
                    (((1,), (1,)), ((), ())),
                    preferred_element_type=jnp.float32,
                )
                e = jnp.exp(s)
                rs = 1.0 / jnp.sum(e, axis=-1, keepdims=True)
                ctx = jnp.dot(
                    e.astype(bf), vq[:, cols], preferred_element_type=jnp.float32
                )
                ctx = (ctx * rs).astype(bf)
                acc = acc + jnp.dot(
                    ctx, wo_ref[h], preferred_element_type=jnp.float32
                )
            part_ref[pl.ds(Q * QROWS, QROWS), :] = acc

        def store_chunk(c, d, data):
            j = lax.rem(c, 4)
            r = c // 4
            b0 = 8 * j + r
            col = pl.ds(d * HALF, HALF)
            out_ref[pl.ds(b0 * 64, 64), col] = data[:64]
            out_ref[pl.ds(b0 * 64 + 256, 64), col] = data[64:]

        def part_q(Q, d):
            return part_ref[pl.ds(Q * QROWS, QROWS), pl.ds(d * HALF, HALF)]

        def accB_z(Z, d):
            return accB[d, pl.ds(Z * CHUNK, CHUNK), :]

        def b_rdma(h, slot, d):
            return pltpu.make_async_remote_copy(
                src_ref=sbB.at[slot, d],
                dst_ref=rbB.at[d, h],
                send_sem=semB_send.at[slot, d],
                recv_sem=semB_recv.at[d, h],
                device_id=(plane_tgt[d],),
                device_id_type=pl.DeviceIdType.MESH,
            )

        compute_quarter(q)
        rdmas = []
        for d in (0, 1):
            sbB[0, d] = part_q(q, d).astype(bf)
            rd = b_rdma(0, 0, d)
            rd.start()
            rdmas.append(rd)
        for h in (1, 2, 3):
            if h == 1:
                compute_quarter(lax.rem(q + 3, 4))
                compute_quarter(lax.rem(q + 1, 4))
            elif h == 2:
                compute_quarter(lax.rem(q + 2, 4))
            for rd in rdmas:
                rd.wait()
            rdmas = []
            if h <= 2:
                slot = h % 2
                for d in (0, 1):
                    Q = lax.rem(q + (h if d else -h) + 4, 4)
                    sbB[slot, d] = (part_q(Q, d) + rbB[d, h - 1]).astype(bf)
                    rd = b_rdma(h, slot, d)
                    rd.start()
                    rdmas.append(rd)
        for d in (0, 1):
            Qo = lax.rem(q + (3 if d else 1), 4)
            accB[d] = part_q(Qo, d) + rbB[d, 2]

        def a_rdma(h, slot, d):
            return pltpu.make_async_remote_copy(
                src_ref=sbA.at[slot, d],
                dst_ref=rbA.at[d, h],
                send_sem=semA_send.at[slot, d],
                recv_sem=semA_recv.at[d, h],
                device_id=(col_tgt[d],),
                device_id_type=pl.DeviceIdType.MESH,
            )

        rdmas = []
        for d in (0, 1):
            sbA[0, d] = accB_z(zi, d).astype(bf)
            rd = a_rdma(0, 0, d)
            rd.start()
            rdmas.append(rd)
        for h in (1, 2, 3):
            for rd in rdmas:
                rd.wait()
            rdmas = []
            if h <= 2:
                slot = h % 2
                for d in (0, 1):
                    Z = lax.rem(zi + (h if d else -h) + 4, 4)
                    sbA[slot, d] = (accB_z(Z, d) + rbA[d, h - 1]).astype(bf)
                    rd = a_rdma(h, slot, d)
                    rd.start()
                    rdmas.append(rd)
        for d in (0, 1):
            Zo = lax.rem(zi + (3 if d else 1), 4)
            Qo = lax.rem(q + (3 if d else 1), 4)
            blk = accB_z(Zo, d) + rbA[d, 2]
            ownA[d] = blk.astype(bf)
            qbuf[d, pl.ds(Zo * CHUNK, CHUNK), :] = blk.astype(bf)
            store_chunk(Qo * 4 + Zo, d, blk)

        def c_rdma(h, d):
            src = ownA.at[d] if h == 0 else rbC.at[d, h - 1]
            return pltpu.make_async_remote_copy(
                src_ref=src,
                dst_ref=rbC.at[d, h],
                send_sem=semC_send.at[h % 2, d],
                recv_sem=semC_recv.at[d, h],
                device_id=(col_tgt[d],),
                device_id_type=pl.DeviceIdType.MESH,
            )

        rdmas = [c_rdma(0, 0), c_rdma(0, 1)]
        for rd in rdmas:
            rd.start()
        for h in (0, 1, 2):
            for rd in rdmas:
                rd.wait()
            if h < 2:
                rdmas = [c_rdma(h + 1, 0), c_rdma(h + 1, 1)]
                for rd in rdmas:
                    rd.start()
            for d in (0, 1):
                Zr = lax.rem(zi + (h if d else -h) + 4, 4)
                Qo = lax.rem(q + (3 if d else 1), 4)
                data = rbC[d, h]
                qbuf[d, pl.ds(Zr * CHUNK, CHUNK), :] = data
                store_chunk(Qo * 4 + Zr, d, data.astype(jnp.float32))

        def d_rdma(h, d):
            src = qbuf.at[d] if h == 0 else rbD.at[d, h - 1]
            return pltpu.make_async_remote_copy(
                src_ref=src,
                dst_ref=rbD.at[d, h],
                send_sem=semD_send.at[h % 2, d],
                recv_sem=semD_recv.at[d, h],
                device_id=(plane_tgt[d],),
                device_id_type=pl.DeviceIdType.MESH,
            )

        rdmas = [d_rdma(0, 0), d_rdma(0, 1)]
        for rd in rdmas:
            rd.start()
        for h in (0, 1, 2):
            for rd in rdmas:
                rd.wait()
            if h < 2:
                rdmas = [d_rdma(h + 1, 0), d_rdma(h + 1, 1)]
                for rd in rdmas:
                    rd.start()
            for d in (0, 1):
                Qr = lax.rem(q + (h if d else -h) + 4, 4)
                for k in range(4):
                    store_chunk(
                        Qr * 4 + k,
                        d,
                        rbD[d, h, pl.ds(k * CHUNK, CHUNK), :].astype(jnp.float32),
                    )

        @functools.partial(pl.run_scoped, sem=pltpu.SemaphoreType.REGULAR)
        def _(sem):
            for nbr in (ipn, ipp, cn, cp):
                pl.semaphore_signal(
                    sem, inc=1, device_id=(nbr,),
                    device_id_type=pl.DeviceIdType.MESH,
                )
            pl.semaphore_wait(sem, 4)

    return pl.pallas_call(
        body,
        out_shape=jax.ShapeDtypeStruct((SQ, D_MODEL), jnp.float32),
        in_specs=[pl.BlockSpec(memory_space=pltpu.VMEM)] * 5,
        out_specs=pl.BlockSpec(memory_space=pltpu.VMEM),
        scratch_shapes=[
            pltpu.VMEM((SQ, D_MODEL), jnp.float32),
            pltpu.VMEM((2, 2, QROWS, HALF), jnp.bfloat16),
            pltpu.VMEM((2, 3, QROWS, HALF), jnp.bfloat16),
            pltpu.VMEM((2, QROWS, HALF), jnp.float32),
            pltpu.VMEM((2, 2, CHUNK, HALF), jnp.bfloat16),
            pltpu.VMEM((2, 3, CHUNK, HALF), jnp.bfloat16),
            pltpu.VMEM((2, CHUNK, HALF), jnp.bfloat16),
            pltpu.VMEM((2, QROWS, HALF), jnp.bfloat16),
            pltpu.VMEM((2, 3, CHUNK, HALF), jnp.bfloat16),
            pltpu.VMEM((2, 3, QROWS, HALF), jnp.bfloat16),
            pltpu.SemaphoreType.DMA((2, 2)),
            pltpu.SemaphoreType.DMA((2, 3)),
            pltpu.SemaphoreType.DMA((2, 2)),
            pltpu.SemaphoreType.DMA((2, 3)),
            pltpu.SemaphoreType.DMA((2, 2)),
            pltpu.SemaphoreType.DMA((2, 3)),
            pltpu.SemaphoreType.DMA((2, 2)),
            pltpu.SemaphoreType.DMA((2, 3)),
        ],
        compiler_params=pltpu.CompilerParams(
            collective_id=0, vmem_limit_bytes=100 * 1024 * 1024
        ),
    )(xb, Wq_b, Kb, Vb, Wo_r)


def kernel(x, Wq, K_ext, V_ext, Wo):
    out = _fused(*_prep(x, Wq, K_ext, V_ext, Wo))
    return out.reshape(1, SQ, D_MODEL)
